# baseline (speedup 1.0000x reference)
import functools

import jax
import jax.numpy as jnp
from jax import lax
from jax.experimental import pallas as pl
from jax.experimental.pallas import tpu as pltpu
from jax.experimental.pallas import tpu_sc as plsc

N = 10000
E = 320000
D = 128
NP = 10240
EP = 327680
NC, NS, L = 2, 16, 16
NW = NC * NS
EW = EP // NW
C = 80
NCH = EW // C
NBK = 16
NBLK = NCH // NBK
TOTCH = EP // C
K0 = 96
K1 = 160
NT = NP // NS


def _mm_bias_kernel(x_ref, w_ref, b_ref, o_ref, *, relu):
    y = jnp.dot(x_ref[...], w_ref[...], preferred_element_type=jnp.float32)
    y = y + b_ref[...]
    if relu:
        y = jnp.maximum(y, 0.0)
    o_ref[...] = y


def _mm_bias(x, w_t, b, relu):
    n, k = x.shape
    m = w_t.shape[1]
    blk = 1000
    return pl.pallas_call(
        functools.partial(_mm_bias_kernel, relu=relu),
        grid=(n // blk,),
        in_specs=[
            pl.BlockSpec((blk, k), lambda i: (i, 0)),
            pl.BlockSpec((k, m), lambda i: (0, 0)),
            pl.BlockSpec((1, m), lambda i: (0, 0)),
        ],
        out_specs=pl.BlockSpec((blk, m), lambda i: (i, 0)),
        out_shape=jax.ShapeDtypeStruct((n, m), jnp.float32),
    )(x, w_t, b.reshape(1, m))


def _rsqrt16(v):
    i = plsc.bitcast(v, jnp.int32)
    i = 0x5F3759DF - lax.shift_right_logical(i, 1)
    y = plsc.bitcast(i, jnp.float32)
    for _ in range(3):
        y = y * (1.5 - 0.5 * v * y * y)
    return y


def _layer_body(h_hbm, hb_hbm, src_hbm, dst_hbm, acc_hbm, dpart_hbm,
                srows, drows, sidx, didx, eebuf, denom, osh,
                ss0, ss1, sd0, sd1):
    cid = lax.axis_index("c")
    tid = lax.axis_index("s")
    wid = tid * NC + cid
    zero16 = jnp.zeros((L,), jnp.float32)
    iota16 = lax.iota(jnp.int32, L)
    ssem = (ss0, ss1)
    dsem = (sd0, sd1)

    def dzero_body(i, c):
        denom[pl.ds(i * L, L)] = zero16
        return c
    lax.fori_loop(0, NP // L, dzero_body, 0)

    def rzero_body(r, c):
        for u in range(D // L):
            srows[0, r, pl.ds(u * L, L)] = zero16
        return c
    lax.fori_loop(0, C, rzero_body, 0)
    for j in range(NT // C):
        pltpu.sync_copy(srows.at[0], osh.at[pl.ds(tid * NT + j * C, C)])
    plsc.subcore_barrier()

    def process_chunk(k, b):
        for g in range(C // L):
            def edge_body(j, carry):
                dot, ns, nd = carry
                e = g * L + j
                da = zero16
                sa = zero16
                na = zero16
                for m2 in range(4):
                    dd32 = drows[b, e, pl.ds(m2 * L, L)]
                    dd = plsc.bitcast(dd32, jnp.bfloat16)
                    d0, d1 = plsc.unpack(dd, format=plsc.PackFormat.INTERLEAVED)
                    for q in range(2):
                        u = m2 * 2 + q
                        sv = srows[b, e, pl.ds(u * L, L)]
                        dv = d0 if q == 0 else d1
                        da = da + sv * dv
                        sa = sa + sv * sv
                        na = na + dv * dv
                m = iota16 == j
                return (jnp.where(m, jnp.sum(da), dot),
                        jnp.where(m, jnp.sum(sa), ns),
                        jnp.where(m, jnp.sum(na), nd))
            dot, ns, nd = lax.fori_loop(0, L, edge_body,
                                        (zero16, zero16, zero16))
            cosv = dot * _rsqrt16(ns + 1e-24) * _rsqrt16(nd + 1e-24)
            eev = jnp.exp(cosv)
            eebuf[pl.ds(g * L, L)] = eev
            plsc.addupdate_scatter(denom, [didx[k, pl.ds(g * L, L)]], eev)

        def scale_body(e, c2):
            a = plsc.load_gather(eebuf, [jnp.full((L,), e, jnp.int32)])
            for u in range(D // L):
                srows[b, e, pl.ds(u * L, L)] = srows[b, e, pl.ds(u * L, L)] * a
            return c2
        lax.fori_loop(0, C, scale_body, 0)
        pltpu.sync_copy(srows.at[b], osh.at[didx.at[k]], add=True)

    def do_blocks(nblocks, start):
        def block_body(nb, c):
            blk0 = start + nb * NBK
            pltpu.sync_copy(src_hbm.at[pl.ds(blk0, NBK)], sidx)
            pltpu.sync_copy(dst_hbm.at[pl.ds(blk0, NBK)], didx)

            def super_body(i2, c2):
                k0 = i2 * 2
                cp = []
                for b in range(2):
                    cp.append((
                        pltpu.async_copy(h_hbm.at[sidx.at[k0 + b]],
                                         srows.at[b], ssem[b]),
                        pltpu.async_copy(hb_hbm.at[didx.at[k0 + b]],
                                         drows.at[b], dsem[b]),
                    ))
                for b in range(2):
                    cp[b][0].wait()
                    cp[b][1].wait()
                    process_chunk(k0 + b, b)
                return c2
            lax.fori_loop(0, NBK // 2, super_body, 0)
            return c
        lax.fori_loop(0, nblocks, block_body, 0)

    @pl.when(cid == 0)
    def _():
        do_blocks(K0 // NBK, tid * (K0 + K1))

    @pl.when(cid == 1)
    def _():
        do_blocks(K1 // NBK, tid * (K0 + K1) + K0)

    plsc.subcore_barrier()
    for j in range(NT // C):
        r0 = tid * NT + j * C
        pltpu.sync_copy(osh.at[pl.ds(r0, C)], srows.at[0])
        pltpu.sync_copy(srows.at[0], acc_hbm.at[cid].at[pl.ds(r0, C)])
    pltpu.sync_copy(denom, dpart_hbm.at[wid])


_sc_layer = pl.kernel(
    _layer_body,
    out_type=(jax.ShapeDtypeStruct((NC, NP, D), jnp.float32),
              jax.ShapeDtypeStruct((NW, NP), jnp.float32)),
    mesh=plsc.VectorSubcoreMesh(core_axis_name="c", subcore_axis_name="s"),
    compiler_params=pltpu.CompilerParams(needs_layout_passes=False, use_tc_tiling_on_sc=False),
    scratch_types=[
        pltpu.VMEM((2, C, D), jnp.float32),
        pltpu.VMEM((2, C, D // 2), jnp.int32),
        pltpu.VMEM((NBK, C), jnp.int32),
        pltpu.VMEM((NBK, C), jnp.int32),
        pltpu.VMEM((C,), jnp.float32),
        pltpu.VMEM((NP,), jnp.float32),
        pltpu.VMEM_SHARED((NP, D), jnp.float32),
        pltpu.SemaphoreType.DMA,
        pltpu.SemaphoreType.DMA,
        pltpu.SemaphoreType.DMA,
        pltpu.SemaphoreType.DMA,
    ],
)


def kernel(x, edge_index, W1, b1, W2, b2):
    pad_e = jnp.full((EP - E,), N, dtype=jnp.int32)
    src3 = jnp.concatenate([edge_index[0], pad_e]).reshape(TOTCH, C)
    dst3 = jnp.concatenate([edge_index[1], pad_e]).reshape(TOTCH, C)

    h = _mm_bias(x, W1.T, b1, relu=True)
    h_pad = jnp.pad(h, ((0, NP - N), (0, 0)))
    row = jnp.arange(NP, dtype=jnp.int32)[:, None]
    blk = jnp.arange(D, dtype=jnp.int32) // 32
    lane = jnp.arange(D, dtype=jnp.int32) % 32
    perm = blk * 32 + jnp.where(lane % 2 == 0, lane // 2, 16 + lane // 2)
    for _ in range(4):
        h_bf = h_pad[:, perm].astype(jnp.bfloat16)
        h_b32 = lax.bitcast_convert_type(h_bf.reshape(NP, D // 2, 2),
                                         jnp.int32)
        acc, dpart = _sc_layer(h_pad, h_b32, src3, dst3)
        out = jax.nn.relu((acc[0] + acc[1])
                          / (dpart.sum(axis=0)[:, None] + 1e-16))
        h_pad = jnp.where(row < N, out, 0.0)
    return _mm_bias(h_pad[:N], W2.T, b2, relu=False)


# speedup vs baseline: 1.1169x; 1.1169x over previous
import functools

import jax
import jax.numpy as jnp
from jax import lax
from jax.experimental import pallas as pl
from jax.experimental.pallas import tpu as pltpu
from jax.experimental.pallas import tpu_sc as plsc

N = 10000
E = 320000
D = 128
NP = 10240
EP = 327680
NC, NS, L = 2, 16, 16
NW = NC * NS
EW = EP // NW
C = 80
NCH = EW // C
NBK = 16
NBLK = NCH // NBK
NT = NP // NS


def _mm_bias_kernel(x_ref, w_ref, b_ref, o_ref, *, relu):
    y = jnp.dot(x_ref[...], w_ref[...], preferred_element_type=jnp.float32)
    y = y + b_ref[...]
    if relu:
        y = jnp.maximum(y, 0.0)
    o_ref[...] = y


def _mm_bias(x, w_t, b, relu):
    n, k = x.shape
    m = w_t.shape[1]
    blk = 1000
    return pl.pallas_call(
        functools.partial(_mm_bias_kernel, relu=relu),
        grid=(n // blk,),
        in_specs=[
            pl.BlockSpec((blk, k), lambda i: (i, 0)),
            pl.BlockSpec((k, m), lambda i: (0, 0)),
            pl.BlockSpec((1, m), lambda i: (0, 0)),
        ],
        out_specs=pl.BlockSpec((blk, m), lambda i: (i, 0)),
        out_shape=jax.ShapeDtypeStruct((n, m), jnp.float32),
    )(x, w_t, b.reshape(1, m))


def _rsqrt16(v):
    i = plsc.bitcast(v, jnp.int32)
    i = 0x5F3759DF - lax.shift_right_logical(i, 1)
    y = plsc.bitcast(i, jnp.float32)
    for _ in range(3):
        y = y * (1.5 - 0.5 * v * y * y)
    return y


def _layer_body(h_hbm, hb_hbm, src_hbm, dst_hbm, acc_hbm, dpart_hbm,
                srows, drows, sidx, didx, eebuf, denom, osh,
                ss0, ss1, sd0, sd1):
    cid = lax.axis_index("c")
    tid = lax.axis_index("s")
    wid = tid * NC + cid
    zero16 = jnp.zeros((L,), jnp.float32)
    iota16 = lax.iota(jnp.int32, L)
    ssem = (ss0, ss1)
    dsem = (sd0, sd1)

    def dzero_body(i, c):
        denom[pl.ds(i * L, L)] = zero16
        return c
    lax.fori_loop(0, NP // L, dzero_body, 0)

    def rzero_body(r, c):
        for u in range(D // L):
            srows[0, r, pl.ds(u * L, L)] = zero16
        return c
    lax.fori_loop(0, C, rzero_body, 0)
    for j in range(NT // C):
        pltpu.sync_copy(srows.at[0], osh.at[pl.ds(tid * NT + j * C, C)])
    plsc.subcore_barrier()

    def process_chunk(k, b):
        for g in range(C // L):
            def edge_body(j, carry):
                dot, ns, nd = carry
                e = g * L + j
                da = zero16
                sa = zero16
                na = zero16
                for m2 in range(4):
                    dd32 = drows[b, e, pl.ds(m2 * L, L)]
                    dd = plsc.bitcast(dd32, jnp.bfloat16)
                    d0, d1 = plsc.unpack(dd, format=plsc.PackFormat.INTERLEAVED)
                    for q in range(2):
                        u = m2 * 2 + q
                        sv = srows[b, e, pl.ds(u * L, L)]
                        dv = d0 if q == 0 else d1
                        da = da + sv * dv
                        sa = sa + sv * sv
                        na = na + dv * dv
                m = iota16 == j
                return (jnp.where(m, jnp.sum(da), dot),
                        jnp.where(m, jnp.sum(sa), ns),
                        jnp.where(m, jnp.sum(na), nd))
            dot, ns, nd = lax.fori_loop(0, L, edge_body,
                                        (zero16, zero16, zero16))
            cosv = dot * _rsqrt16(ns + 1e-24) * _rsqrt16(nd + 1e-24)
            eev = jnp.exp(cosv)
            eebuf[pl.ds(g * L, L)] = eev
            plsc.addupdate_scatter(denom, [didx[k, pl.ds(g * L, L)]], eev)

        def scale_body(e, c2):
            a = plsc.load_gather(eebuf, [jnp.full((L,), e, jnp.int32)])
            for u in range(D // L):
                srows[b, e, pl.ds(u * L, L)] = srows[b, e, pl.ds(u * L, L)] * a
            return c2
        lax.fori_loop(0, C, scale_body, 0)
        pltpu.sync_copy(srows.at[b], osh.at[didx.at[k]], add=True)

    def block_body(nb, c):
        blk0 = nb * NBK
        pltpu.sync_copy(src_hbm.at[wid].at[pl.ds(blk0, NBK)], sidx)
        pltpu.sync_copy(dst_hbm.at[wid].at[pl.ds(blk0, NBK)], didx)

        def super_body(i2, c2):
            k0 = i2 * 2
            cp = []
            for b in range(2):
                cp.append((
                    pltpu.async_copy(h_hbm.at[sidx.at[k0 + b]], srows.at[b],
                                     ssem[b]),
                    pltpu.async_copy(hb_hbm.at[didx.at[k0 + b]], drows.at[b],
                                     dsem[b]),
                ))
            for b in range(2):
                cp[b][0].wait()
                cp[b][1].wait()
                process_chunk(k0 + b, b)
            return c2
        lax.fori_loop(0, NBK // 2, super_body, 0)
        return c
    lax.fori_loop(0, NBLK, block_body, 0)

    plsc.subcore_barrier()
    for j in range(NT // C):
        r0 = tid * NT + j * C
        pltpu.sync_copy(osh.at[pl.ds(r0, C)], srows.at[0])
        pltpu.sync_copy(srows.at[0], acc_hbm.at[cid].at[pl.ds(r0, C)])
    pltpu.sync_copy(denom, dpart_hbm.at[wid])


_sc_layer = pl.kernel(
    _layer_body,
    out_type=(jax.ShapeDtypeStruct((NC, NP, D), jnp.float32),
              jax.ShapeDtypeStruct((NW, NP), jnp.float32)),
    mesh=plsc.VectorSubcoreMesh(core_axis_name="c", subcore_axis_name="s"),
    compiler_params=pltpu.CompilerParams(needs_layout_passes=False, use_tc_tiling_on_sc=False),
    scratch_types=[
        pltpu.VMEM((2, C, D), jnp.float32),
        pltpu.VMEM((2, C, D // 2), jnp.int32),
        pltpu.VMEM((NBK, C), jnp.int32),
        pltpu.VMEM((NBK, C), jnp.int32),
        pltpu.VMEM((C,), jnp.float32),
        pltpu.VMEM((NP,), jnp.float32),
        pltpu.VMEM_SHARED((NP, D), jnp.float32),
        pltpu.SemaphoreType.DMA,
        pltpu.SemaphoreType.DMA,
        pltpu.SemaphoreType.DMA,
        pltpu.SemaphoreType.DMA,
    ],
)


def kernel(x, edge_index, W1, b1, W2, b2):
    pad_e = jnp.full((EP - E,), N, dtype=jnp.int32)
    src3 = jnp.concatenate([edge_index[0], pad_e]).reshape(NW, NCH, C)
    dst3 = jnp.concatenate([edge_index[1], pad_e]).reshape(NW, NCH, C)

    h = _mm_bias(x, W1.T, b1, relu=True)
    h_pad = jnp.pad(h, ((0, NP - N), (0, 0)))
    row = jnp.arange(NP, dtype=jnp.int32)[:, None]
    blk = jnp.arange(D, dtype=jnp.int32) // 32
    lane = jnp.arange(D, dtype=jnp.int32) % 32
    perm = blk * 32 + jnp.where(lane % 2 == 0, lane // 2, 16 + lane // 2)
    for _ in range(4):
        h_bf = h_pad[:, perm].astype(jnp.bfloat16)
        h_b32 = lax.bitcast_convert_type(h_bf.reshape(NP, D // 2, 2),
                                         jnp.int32)
        acc, dpart = _sc_layer(h_pad, h_b32, src3, dst3)
        out = jax.nn.relu((acc[0] + acc[1])
                          / (dpart.sum(axis=0)[:, None] + 1e-16))
        h_pad = jnp.where(row < N, out, 0.0)
    return _mm_bias(h_pad[:N], W2.T, b2, relu=False)
